# Initial kernel scaffold; baseline (speedup 1.0000x reference)
import jax
import jax.numpy as jnp
from jax.experimental import pallas as pl


def kernel(x, edge_index, W, b):
    raise NotImplementedError("write your pallas kernel here")


# speedup vs baseline: 15.6943x; 15.6943x over previous
import functools

import jax
import jax.numpy as jnp
from jax import lax
from jax.experimental import pallas as pl
from jax.experimental.pallas import tpu as pltpu
from jax.experimental.pallas import tpu_sc as plsc

N = 10000
E = 320000
D = 128

NC = 2
NS = 16
NP = 10240
ROWS_PER_TILE = NP // NS
CHUNK = 80
EDGES_PER_SC = E // NC
EDGES_PER_TILE = EDGES_PER_SC // NS
NCHUNKS = EDGES_PER_TILE // CHUNK

_mesh = plsc.VectorSubcoreMesh(core_axis_name="c", subcore_axis_name="s")


def _zero_vec(ref, n16):
    z = jnp.zeros((16,), jnp.float32)
    for j in range(n16):
        ref[pl.ds(j * 16, 16)] = z


@functools.partial(
    pl.kernel,
    out_type=jax.ShapeDtypeStruct((NC, NP), jnp.float32),
    mesh=_mesh,
    scratch_types=[
        pltpu.VMEM_SHARED((NP,), jnp.float32),
        pltpu.VMEM((CHUNK,), jnp.int32),
        pltpu.VMEM((CHUNK,), jnp.float32),
        pltpu.VMEM((CHUNK,), jnp.float32),
    ],
)
def _deg_kernel(dst_hbm, out_hbm, dacc, dstv, ones_v, zeros_v):
    cid = lax.axis_index("c")
    sid = lax.axis_index("s")

    one = jnp.ones((16,), jnp.float32)
    for j in range(CHUNK // 16):
        ones_v[pl.ds(j * 16, 16)] = one
    _zero_vec(zeros_v, CHUNK // 16)

    for k in range(ROWS_PER_TILE // CHUNK):
        pltpu.sync_copy(zeros_v, dacc.at[pl.ds(sid * ROWS_PER_TILE + k * CHUNK, CHUNK)])
    plsc.subcore_barrier()

    tile_base = cid * EDGES_PER_SC + sid * EDGES_PER_TILE

    def body(c, carry):
        base = tile_base + c * CHUNK
        pltpu.sync_copy(dst_hbm.at[pl.ds(base, CHUNK)], dstv)
        pltpu.sync_copy(ones_v, dacc.at[dstv], add=True)
        return carry

    lax.fori_loop(0, NCHUNKS, body, 0)
    plsc.subcore_barrier()

    row0 = sid * ROWS_PER_TILE
    pltpu.sync_copy(dacc.at[pl.ds(row0, ROWS_PER_TILE)],
                    out_hbm.at[cid, pl.ds(row0, ROWS_PER_TILE)])


@functools.partial(
    pl.kernel,
    out_type=jax.ShapeDtypeStruct((NC, NP, D), jnp.float32),
    mesh=_mesh,
    scratch_types=[
        pltpu.VMEM_SHARED((NP, D), jnp.float32),
        pltpu.VMEM((CHUNK,), jnp.int32),
        pltpu.VMEM((CHUNK,), jnp.int32),
        pltpu.VMEM((CHUNK, D), jnp.float32),
        pltpu.SemaphoreType.DMA,
    ],
)
def _agg_kernel(hs_hbm, src_hbm, dst_hbm, out_hbm, acc, src_v, dst_v, rows_v, sem):
    cid = lax.axis_index("c")
    sid = lax.axis_index("s")

    z = jnp.zeros((16,), jnp.float32)

    def zbody(i, carry):
        r = i // (D // 16)
        j = i % (D // 16)
        rows_v[r, pl.ds(j * 16, 16)] = z
        return carry

    lax.fori_loop(0, CHUNK * (D // 16), zbody, 0)
    for k in range(ROWS_PER_TILE // CHUNK):
        pltpu.sync_copy(rows_v, acc.at[pl.ds(sid * ROWS_PER_TILE + k * CHUNK, CHUNK)])
    plsc.subcore_barrier()

    tile_base = cid * EDGES_PER_SC + sid * EDGES_PER_TILE

    def body(c, carry):
        base = tile_base + c * CHUNK
        pltpu.sync_copy(src_hbm.at[pl.ds(base, CHUNK)], src_v)
        pltpu.sync_copy(dst_hbm.at[pl.ds(base, CHUNK)], dst_v)
        pltpu.async_copy(hs_hbm.at[src_v], rows_v, sem).wait()
        pltpu.sync_copy(rows_v, acc.at[dst_v], add=True)
        return carry

    lax.fori_loop(0, NCHUNKS, body, 0)
    plsc.subcore_barrier()

    row0 = sid * ROWS_PER_TILE
    pltpu.sync_copy(acc.at[pl.ds(row0, ROWS_PER_TILE)],
                    out_hbm.at[cid, pl.ds(row0, ROWS_PER_TILE)])


_BN = 1000


def _matmul_body(x_ref, w_ref, dv_ref, o_ref):
    o_ref[...] = jnp.dot(x_ref[...], w_ref[...],
                         preferred_element_type=jnp.float32) * dv_ref[...]


def _out_body(x_ref, b_ref, dv_ref, s0_ref, s1_ref, hs_ref, o_ref):
    s = s0_ref[...] + s1_ref[...] + hs_ref[...]
    o_ref[...] = x_ref[...] + b_ref[...] + dv_ref[...] * s


def kernel(x, edge_index, W, b):
    src = edge_index[0]
    dst = edge_index[1]

    dp = _deg_kernel(dst)
    deg = 1.0 + dp[0, :N] + dp[1, :N]
    dinvb = jnp.broadcast_to(lax.rsqrt(deg)[:, None], (N, D))

    grid = (N // _BN,)
    row_spec = pl.BlockSpec((_BN, D), lambda i: (i, 0))
    full_spec = pl.BlockSpec((D, D), lambda i: (0, 0))
    b_spec = pl.BlockSpec((1, D), lambda i: (0, 0))

    hs = pl.pallas_call(
        _matmul_body,
        grid=grid,
        in_specs=[row_spec, full_spec, row_spec],
        out_specs=row_spec,
        out_shape=jax.ShapeDtypeStruct((N, D), jnp.float32),
    )(x, W, dinvb)

    sp = _agg_kernel(hs, src, dst)

    out = pl.pallas_call(
        _out_body,
        grid=grid,
        in_specs=[row_spec, b_spec, row_spec, row_spec, row_spec, row_spec],
        out_specs=row_spec,
        out_shape=jax.ShapeDtypeStruct((N, D), jnp.float32),
    )(x, b.reshape(1, D), dinvb, sp[0, :N], sp[1, :N], hs)

    return out
